# initial kernel scaffold (unmeasured)
import jax
import jax.numpy as jnp
from jax import lax
from jax.experimental import pallas as pl
from jax.experimental.pallas import tpu as pltpu

N_DEV = 32


def _ring_all_reduce(x, collective_id):
    m, n = x.shape
    chunk = m // N_DEV
    n_hops = N_DEV - 1

    def body(x_ref, out_ref, comm_ref, send_sems, recv_sems):
        my = lax.axis_index("i")
        right = lax.rem(my + 1, N_DEV)

        out_ref[:, :] = x_ref[:, :]

        def rs_step(h, carry):
            c_send = lax.rem(my - h + 2 * N_DEV, N_DEV)
            c_recv = lax.rem(my - h - 1 + 2 * N_DEV, N_DEV)
            rdma = pltpu.make_async_remote_copy(
                src_ref=out_ref.at[pl.ds(c_send * chunk, chunk), :],
                dst_ref=comm_ref.at[h],
                send_sem=send_sems.at[h],
                recv_sem=recv_sems.at[h],
                device_id=(right,),
                device_id_type=pl.DeviceIdType.MESH,
            )
            rdma.start()
            rdma.wait()
            out_ref[pl.ds(c_recv * chunk, chunk), :] += comm_ref[h]
            return carry

        lax.fori_loop(0, n_hops, rs_step, 0)

        def ag_step(h, carry):
            c_send = lax.rem(my + 1 - h + 2 * N_DEV, N_DEV)
            c_recv = lax.rem(my - h + 2 * N_DEV, N_DEV)
            s = n_hops + h
            rdma = pltpu.make_async_remote_copy(
                src_ref=out_ref.at[pl.ds(c_send * chunk, chunk), :],
                dst_ref=comm_ref.at[s],
                send_sem=send_sems.at[s],
                recv_sem=recv_sems.at[s],
                device_id=(right,),
                device_id_type=pl.DeviceIdType.MESH,
            )
            rdma.start()
            rdma.wait()
            out_ref[pl.ds(c_recv * chunk, chunk), :] = comm_ref[s]
            return carry

        lax.fori_loop(0, n_hops, ag_step, 0)

    return pl.pallas_call(
        body,
        out_shape=jax.ShapeDtypeStruct((m, n), x.dtype),
        in_specs=[pl.BlockSpec(memory_space=pltpu.VMEM)],
        out_specs=pl.BlockSpec(memory_space=pltpu.VMEM),
        scratch_shapes=[
            pltpu.VMEM((2 * n_hops, chunk, n), x.dtype),
            pltpu.SemaphoreType.DMA((2 * n_hops,)),
            pltpu.SemaphoreType.DMA((2 * n_hops,)),
        ],
        compiler_params=pltpu.CompilerParams(collective_id=collective_id),
    )(x)


def kernel(x, Wq, Wk, Wv, Wo, t_emb, W_mod, W_ff1, W_ff2):
    B, S, D = x.shape
    eps = 1e-5
    Dh = 96
    Hq = Wq.shape[1] // Dh

    mod = t_emb @ W_mod
    sa, sha, ga, sm, shm, gm = jnp.split(mod, 6, axis=-1)

    def ln(h):
        mu = h.mean(axis=-1, keepdims=True)
        var = h.var(axis=-1, keepdims=True)
        return (h - mu) / jnp.sqrt(var + eps)

    x0 = x
    xa = ln(x0) * (1.0 + sa[:, None, :]) + sha[:, None, :]

    Q = (xa @ Wq).reshape(B, S, Hq, Dh)
    K = (xa @ Wk).reshape(B, S, Hq, Dh)
    V = (xa @ Wv).reshape(B, S, Hq, Dh)
    scores = jnp.einsum("bihd,bjhd->bhij", Q, K) * 0.10206207261596577
    p = jax.nn.softmax(scores, axis=-1)
    attn = jnp.einsum("bhij,bjhd->bihd", p, V).reshape(B, S, Hq * Dh)

    attn_partial = attn @ Wo
    attn_out = _ring_all_reduce(attn_partial.reshape(B * S, D), 0)
    attn_out = attn_out.reshape(B, S, D)

    x1 = x0 + ga[:, None, :] * attn_out
    xm = ln(x1) * (1.0 + sm[:, None, :]) + shm[:, None, :]
    h = xm @ W_ff1
    h = h * jax.nn.sigmoid(h)
    ff_partial = h @ W_ff2
    ff_out = _ring_all_reduce(ff_partial.reshape(B * S, D), 1)
    ff_out = ff_out.reshape(B, S, D)

    return x1 + gm[:, None, :] * ff_out


# baseline (device time: 404698 ns/iter reference)
import jax
import jax.numpy as jnp
from jax import lax
from jax.experimental import pallas as pl
from jax.experimental.pallas import tpu as pltpu

N_DEV = 32


def _ring_all_reduce(x, collective_id):
    m, n = x.shape
    chunk = m // N_DEV
    n_hops = N_DEV - 1

    def body(x_ref, out_ref, comm_ref, send_sems, recv_sems):
        my = lax.axis_index("i")
        right = lax.rem(my + 1, N_DEV)

        out_ref[:, :] = x_ref[:, :]

        def rs_step(h, carry):
            c_send = lax.rem(my - h + 2 * N_DEV, N_DEV)
            c_recv = lax.rem(my - h - 1 + 2 * N_DEV, N_DEV)
            rdma = pltpu.make_async_remote_copy(
                src_ref=out_ref.at[pl.ds(c_send * chunk, chunk), :],
                dst_ref=comm_ref.at[h],
                send_sem=send_sems.at[h],
                recv_sem=recv_sems.at[h],
                device_id=(right,),
                device_id_type=pl.DeviceIdType.MESH,
            )
            rdma.start()
            rdma.wait()
            out_ref[pl.ds(c_recv * chunk, chunk), :] += comm_ref[h]
            return carry

        lax.fori_loop(0, n_hops, rs_step, 0)

        def ag_step(h, carry):
            c_send = lax.rem(my + 1 - h + 2 * N_DEV, N_DEV)
            c_recv = lax.rem(my - h + 2 * N_DEV, N_DEV)
            s = n_hops + h
            rdma = pltpu.make_async_remote_copy(
                src_ref=out_ref.at[pl.ds(c_send * chunk, chunk), :],
                dst_ref=comm_ref.at[s],
                send_sem=send_sems.at[s],
                recv_sem=recv_sems.at[s],
                device_id=(right,),
                device_id_type=pl.DeviceIdType.MESH,
            )
            rdma.start()
            rdma.wait()
            out_ref[pl.ds(c_recv * chunk, chunk), :] = comm_ref[s]
            return carry

        lax.fori_loop(0, n_hops, ag_step, 0)

    return pl.pallas_call(
        body,
        out_shape=jax.ShapeDtypeStruct((m, n), x.dtype),
        in_specs=[pl.BlockSpec(memory_space=pltpu.VMEM)],
        out_specs=pl.BlockSpec(memory_space=pltpu.VMEM),
        scratch_shapes=[
            pltpu.VMEM((2 * n_hops, chunk, n), x.dtype),
            pltpu.SemaphoreType.DMA((2 * n_hops,)),
            pltpu.SemaphoreType.DMA((2 * n_hops,)),
        ],
    )(x)


def kernel(x, Wq, Wk, Wv, Wo, t_emb, W_mod, W_ff1, W_ff2):
    B, S, D = x.shape
    eps = 1e-5
    Dh = 96
    Hq = Wq.shape[1] // Dh

    mod = t_emb @ W_mod
    sa, sha, ga, sm, shm, gm = jnp.split(mod, 6, axis=-1)

    def ln(h):
        mu = h.mean(axis=-1, keepdims=True)
        var = h.var(axis=-1, keepdims=True)
        return (h - mu) / jnp.sqrt(var + eps)

    x0 = x
    xa = ln(x0) * (1.0 + sa[:, None, :]) + sha[:, None, :]

    Q = (xa @ Wq).reshape(B, S, Hq, Dh)
    K = (xa @ Wk).reshape(B, S, Hq, Dh)
    V = (xa @ Wv).reshape(B, S, Hq, Dh)
    scores = jnp.einsum("bihd,bjhd->bhij", Q, K) * 0.10206207261596577
    p = jax.nn.softmax(scores, axis=-1)
    attn = jnp.einsum("bhij,bjhd->bihd", p, V).reshape(B, S, Hq * Dh)

    attn_partial = attn @ Wo
    attn_out = _ring_all_reduce(attn_partial.reshape(B * S, D), 0)
    attn_out = attn_out.reshape(B, S, D)

    x1 = x0 + ga[:, None, :] * attn_out
    xm = ln(x1) * (1.0 + sm[:, None, :]) + shm[:, None, :]
    h = xm @ W_ff1
    h = h * jax.nn.sigmoid(h)
    ff_partial = h @ W_ff2
    ff_out = _ring_all_reduce(ff_partial.reshape(B * S, D), 1)
    ff_out = ff_out.reshape(B, S, D)

    return x1 + gm[:, None, :] * ff_out


# device time: 121418 ns/iter; 3.3331x vs baseline; 3.3331x over previous
import jax
import jax.numpy as jnp
from jax import lax
from jax.experimental import pallas as pl
from jax.experimental.pallas import tpu as pltpu

N_DEV = 32


WIRE_DTYPE = jnp.bfloat16


def _ring_all_reduce(x, collective_id):
    m, n = x.shape
    chunk = m // N_DEV
    xw = x.astype(WIRE_DTYPE)

    def body(x_ref, xw_ref, out_ref, rs_buf, ag_buf,
             rs_send, rs_recv, ag_send, ag_recv):
        my = lax.axis_index("i")

        rs_sends = []
        for k in range(1, N_DEV):
            peer = lax.rem(my + k, N_DEV)
            rdma = pltpu.make_async_remote_copy(
                src_ref=xw_ref.at[pl.ds(peer * chunk, chunk), :],
                dst_ref=rs_buf.at[my],
                send_sem=rs_send.at[k],
                recv_sem=rs_recv.at[my],
                device_id=(peer,),
                device_id_type=pl.DeviceIdType.MESH,
            )
            rdma.start()
            rs_sends.append(rdma)

        out_ref[pl.ds(my * chunk, chunk), :] = x_ref[pl.ds(my * chunk, chunk), :]
        for k in range(1, N_DEV):
            src = lax.rem(my + k, N_DEV)
            recv = pltpu.make_async_remote_copy(
                src_ref=rs_buf.at[src],
                dst_ref=rs_buf.at[src],
                send_sem=rs_send.at[0],
                recv_sem=rs_recv.at[src],
                device_id=(my,),
                device_id_type=pl.DeviceIdType.MESH,
            )
            recv.wait_recv()
            out_ref[pl.ds(my * chunk, chunk), :] += rs_buf[src].astype(x.dtype)

        ag_buf[my] = out_ref[pl.ds(my * chunk, chunk), :].astype(WIRE_DTYPE)
        ag_sends = []
        for k in range(1, N_DEV):
            peer = lax.rem(my + k, N_DEV)
            rdma = pltpu.make_async_remote_copy(
                src_ref=ag_buf.at[my],
                dst_ref=ag_buf.at[my],
                send_sem=ag_send.at[k],
                recv_sem=ag_recv.at[my],
                device_id=(peer,),
                device_id_type=pl.DeviceIdType.MESH,
            )
            rdma.start()
            ag_sends.append(rdma)

        for k in range(1, N_DEV):
            src = lax.rem(my + k, N_DEV)
            recv = pltpu.make_async_remote_copy(
                src_ref=ag_buf.at[src],
                dst_ref=ag_buf.at[src],
                send_sem=ag_send.at[0],
                recv_sem=ag_recv.at[src],
                device_id=(my,),
                device_id_type=pl.DeviceIdType.MESH,
            )
            recv.wait_recv()
            out_ref[pl.ds(src * chunk, chunk), :] = ag_buf[src].astype(x.dtype)

        for d in rs_sends + ag_sends:
            d.wait_send()

    return pl.pallas_call(
        body,
        out_shape=jax.ShapeDtypeStruct((m, n), x.dtype),
        in_specs=[pl.BlockSpec(memory_space=pltpu.VMEM)] * 2,
        out_specs=pl.BlockSpec(memory_space=pltpu.VMEM),
        scratch_shapes=[
            pltpu.VMEM((N_DEV, chunk, n), WIRE_DTYPE),
            pltpu.VMEM((N_DEV, chunk, n), WIRE_DTYPE),
            pltpu.SemaphoreType.DMA((N_DEV,)),
            pltpu.SemaphoreType.DMA((N_DEV,)),
            pltpu.SemaphoreType.DMA((N_DEV,)),
            pltpu.SemaphoreType.DMA((N_DEV,)),
        ],
    )(x, xw)


def kernel(x, Wq, Wk, Wv, Wo, t_emb, W_mod, W_ff1, W_ff2):
    B, S, D = x.shape
    eps = 1e-5
    Dh = 96
    Hq = Wq.shape[1] // Dh

    mod = t_emb @ W_mod
    sa, sha, ga, sm, shm, gm = jnp.split(mod, 6, axis=-1)

    def ln(h):
        mu = h.mean(axis=-1, keepdims=True)
        var = h.var(axis=-1, keepdims=True)
        return (h - mu) / jnp.sqrt(var + eps)

    x0 = x
    xa = ln(x0) * (1.0 + sa[:, None, :]) + sha[:, None, :]

    Q = (xa @ Wq).reshape(B, S, Hq, Dh)
    K = (xa @ Wk).reshape(B, S, Hq, Dh)
    V = (xa @ Wv).reshape(B, S, Hq, Dh)
    scores = jnp.einsum("bihd,bjhd->bhij", Q, K) * 0.10206207261596577
    p = jax.nn.softmax(scores, axis=-1)
    attn = jnp.einsum("bhij,bjhd->bihd", p, V).reshape(B, S, Hq * Dh)

    attn_partial = attn @ Wo
    attn_out = _ring_all_reduce(attn_partial.reshape(B * S, D), 0)
    attn_out = attn_out.reshape(B, S, D)

    x1 = x0 + ga[:, None, :] * attn_out
    xm = ln(x1) * (1.0 + sm[:, None, :]) + shm[:, None, :]
    h = xm @ W_ff1
    h = h * jax.nn.sigmoid(h)
    ff_partial = h @ W_ff2
    ff_out = _ring_all_reduce(ff_partial.reshape(B * S, D), 1)
    ff_out = ff_out.reshape(B, S, D)

    return x1 + gm[:, None, :] * ff_out


# device time: 120865 ns/iter; 3.3483x vs baseline; 1.0046x over previous
import jax
import jax.numpy as jnp
from jax import lax
from jax.experimental import pallas as pl
from jax.experimental.pallas import tpu as pltpu

N_DEV = 32


WIRE_DTYPE = jnp.bfloat16


def _ring_all_reduce(x, collective_id):
    m, n = x.shape
    chunk = m // N_DEV
    xw = x.astype(WIRE_DTYPE)

    def body(x_ref, xw_ref, out_ref, rs_buf, ag_buf,
             rs_send, rs_recv, ag_send, ag_recv):
        my = lax.axis_index("i")

        rs_sends = []
        for k in range(1, N_DEV):
            peer = lax.rem(my + k, N_DEV)
            rdma = pltpu.make_async_remote_copy(
                src_ref=xw_ref.at[pl.ds(peer * chunk, chunk), :],
                dst_ref=rs_buf.at[my],
                send_sem=rs_send.at[k],
                recv_sem=rs_recv.at[my],
                device_id=(peer,),
                device_id_type=pl.DeviceIdType.MESH,
            )
            rdma.start()
            rs_sends.append(rdma)

        rs_buf[my] = jnp.zeros((chunk, n), WIRE_DTYPE)
        for k in range(1, N_DEV):
            src = lax.rem(my + k, N_DEV)
            recv = pltpu.make_async_remote_copy(
                src_ref=rs_buf.at[src],
                dst_ref=rs_buf.at[src],
                send_sem=rs_send.at[0],
                recv_sem=rs_recv.at[src],
                device_id=(my,),
                device_id_type=pl.DeviceIdType.MESH,
            )
            recv.wait_recv()
        acc = x_ref[pl.ds(my * chunk, chunk), :] + jnp.sum(
            rs_buf[...].astype(x.dtype), axis=0
        )
        out_ref[pl.ds(my * chunk, chunk), :] = acc

        ag_buf[my] = acc.astype(WIRE_DTYPE)
        ag_sends = []
        for k in range(1, N_DEV):
            peer = lax.rem(my + k, N_DEV)
            rdma = pltpu.make_async_remote_copy(
                src_ref=ag_buf.at[my],
                dst_ref=ag_buf.at[my],
                send_sem=ag_send.at[k],
                recv_sem=ag_recv.at[my],
                device_id=(peer,),
                device_id_type=pl.DeviceIdType.MESH,
            )
            rdma.start()
            ag_sends.append(rdma)

        for k in range(1, N_DEV):
            src = lax.rem(my + k, N_DEV)
            recv = pltpu.make_async_remote_copy(
                src_ref=ag_buf.at[src],
                dst_ref=ag_buf.at[src],
                send_sem=ag_send.at[0],
                recv_sem=ag_recv.at[src],
                device_id=(my,),
                device_id_type=pl.DeviceIdType.MESH,
            )
            recv.wait_recv()
        out_ref[:, :] = ag_buf[...].reshape(m, n).astype(x.dtype)
        out_ref[pl.ds(my * chunk, chunk), :] = acc

        for d in rs_sends + ag_sends:
            d.wait_send()

    return pl.pallas_call(
        body,
        out_shape=jax.ShapeDtypeStruct((m, n), x.dtype),
        in_specs=[pl.BlockSpec(memory_space=pltpu.VMEM)] * 2,
        out_specs=pl.BlockSpec(memory_space=pltpu.VMEM),
        scratch_shapes=[
            pltpu.VMEM((N_DEV, chunk, n), WIRE_DTYPE),
            pltpu.VMEM((N_DEV, chunk, n), WIRE_DTYPE),
            pltpu.SemaphoreType.DMA((N_DEV,)),
            pltpu.SemaphoreType.DMA((N_DEV,)),
            pltpu.SemaphoreType.DMA((N_DEV,)),
            pltpu.SemaphoreType.DMA((N_DEV,)),
        ],
    )(x, xw)


def kernel(x, Wq, Wk, Wv, Wo, t_emb, W_mod, W_ff1, W_ff2):
    B, S, D = x.shape
    eps = 1e-5
    Dh = 96
    Hq = Wq.shape[1] // Dh

    mod = t_emb @ W_mod
    sa, sha, ga, sm, shm, gm = jnp.split(mod, 6, axis=-1)

    def ln(h):
        mu = h.mean(axis=-1, keepdims=True)
        var = h.var(axis=-1, keepdims=True)
        return (h - mu) / jnp.sqrt(var + eps)

    x0 = x
    xa = ln(x0) * (1.0 + sa[:, None, :]) + sha[:, None, :]

    Q = (xa @ Wq).reshape(B, S, Hq, Dh)
    K = (xa @ Wk).reshape(B, S, Hq, Dh)
    V = (xa @ Wv).reshape(B, S, Hq, Dh)
    scores = jnp.einsum("bihd,bjhd->bhij", Q, K) * 0.10206207261596577
    p = jax.nn.softmax(scores, axis=-1)
    attn = jnp.einsum("bhij,bjhd->bihd", p, V).reshape(B, S, Hq * Dh)

    attn_partial = attn @ Wo
    attn_out = _ring_all_reduce(attn_partial.reshape(B * S, D), 0)
    attn_out = attn_out.reshape(B, S, D)

    x1 = x0 + ga[:, None, :] * attn_out
    xm = ln(x1) * (1.0 + sm[:, None, :]) + shm[:, None, :]
    h = xm @ W_ff1
    h = h * jax.nn.sigmoid(h)
    ff_partial = h @ W_ff2
    ff_out = _ring_all_reduce(ff_partial.reshape(B * S, D), 1)
    ff_out = ff_out.reshape(B, S, D)

    return x1 + gm[:, None, :] * ff_out


# device time: 112542 ns/iter; 3.5960x vs baseline; 1.0740x over previous
import jax
import jax.numpy as jnp
from jax import lax
from jax.experimental import pallas as pl
from jax.experimental.pallas import tpu as pltpu

N_DEV = 32
WIRE = jnp.bfloat16
F32 = jnp.float32


def _fused_post_attn(attn_partial, x0, mods, W_ff1, W_ff2):
    m, n = attn_partial.shape
    chunk = m // N_DEV
    half = m // 2
    eps = 1e-5
    ap_w = attn_partial.astype(WIRE)

    def body(ap_ref, x0_ref, mods_ref, w1_ref, w2_ref, out_ref,
             rs1_buf, ag1_buf, rs2_src, rs2_buf, ag2_buf, x1_buf,
             rs1_send, rs1_recv, ag1_send, ag1_recv,
             rs2_send, rs2_recv, ag2_send, ag2_recv):
        my = lax.axis_index("i")
        my_lo = my * chunk

        def send(src_ref, slot_remote, sem_send, sem_recv, k, peer):
            rdma = pltpu.make_async_remote_copy(
                src_ref=src_ref,
                dst_ref=slot_remote,
                send_sem=sem_send.at[k],
                recv_sem=sem_recv.at[my],
                device_id=(peer,),
                device_id_type=pl.DeviceIdType.MESH,
            )
            rdma.start()
            return rdma

        def wait_recvs(buf, sem_recv, sem_send):
            for k in range(1, N_DEV):
                src = lax.rem(my + k, N_DEV)
                pltpu.make_async_remote_copy(
                    src_ref=buf.at[src], dst_ref=buf.at[src],
                    send_sem=sem_send.at[0], recv_sem=sem_recv.at[src],
                    device_id=(my,), device_id_type=pl.DeviceIdType.MESH,
                ).wait_recv()

        in_flight = []

        rs1_buf[my] = ap_ref[pl.ds(my_lo, chunk), :]
        for k in range(1, N_DEV):
            peer = lax.rem(my + k, N_DEV)
            in_flight.append(send(
                ap_ref.at[pl.ds(peer * chunk, chunk), :], rs1_buf.at[my],
                rs1_send, rs1_recv, k, peer))
        wait_recvs(rs1_buf, rs1_recv, rs1_send)
        acc1 = jnp.sum(rs1_buf[...].astype(F32), axis=0)

        ag1_buf[my] = acc1.astype(WIRE)
        for k in range(1, N_DEV):
            peer = lax.rem(my + k, N_DEV)
            in_flight.append(send(
                ag1_buf.at[my], ag1_buf.at[my], ag1_send, ag1_recv, k, peer))
        wait_recvs(ag1_buf, ag1_recv, ag1_send)
        a1 = ag1_buf[...].reshape(m, n).astype(F32)

        def ffn_half(lo, ga, sm, shm):
            x1 = x0_ref[lo:lo + half, :] + ga * a1[lo:lo + half, :]
            x1_buf[lo:lo + half, :] = x1
            mu = jnp.mean(x1, axis=-1, keepdims=True)
            var = jnp.mean((x1 - mu) * (x1 - mu), axis=-1, keepdims=True)
            xm = ((x1 - mu) / jnp.sqrt(var + eps)) * (1.0 + sm) + shm
            h = jnp.dot(xm, w1_ref[...], preferred_element_type=F32)
            h = h * (1.0 / (1.0 + jnp.exp(-h)))
            ffp = jnp.dot(h, w2_ref[...], preferred_element_type=F32)
            c0 = lo // chunk
            rs2_src[c0:c0 + half // chunk] = ffp.reshape(
                half // chunk, chunk, n).astype(WIRE)

        ffn_half(0, mods_ref[0:1, :], mods_ref[2:3, :], mods_ref[4:5, :])
        ffn_half(half, mods_ref[1:2, :], mods_ref[3:4, :], mods_ref[5:6, :])

        rs2_buf[my] = rs2_src[my]
        for k in range(1, N_DEV):
            peer = lax.rem(my + k, N_DEV)
            in_flight.append(send(
                rs2_src.at[peer], rs2_buf.at[my], rs2_send, rs2_recv, k, peer))
        wait_recvs(rs2_buf, rs2_recv, rs2_send)
        acc2 = jnp.sum(rs2_buf[...].astype(F32), axis=0)

        ag2_buf[my] = acc2.astype(WIRE)
        for k in range(1, N_DEV):
            peer = lax.rem(my + k, N_DEV)
            in_flight.append(send(
                ag2_buf.at[my], ag2_buf.at[my], ag2_send, ag2_recv, k, peer))
        wait_recvs(ag2_buf, ag2_recv, ag2_send)

        ff = ag2_buf[...].reshape(m, n).astype(F32)
        out_ref[0:half, :] = x1_buf[0:half, :] + mods_ref[6:7, :] * ff[0:half, :]
        out_ref[half:m, :] = x1_buf[half:m, :] + mods_ref[7:8, :] * ff[half:m, :]
        gm_my = jnp.where(my < (half // chunk), mods_ref[6:7, :], mods_ref[7:8, :])
        out_ref[pl.ds(my_lo, chunk), :] = (
            x1_buf[pl.ds(my_lo, chunk), :] + gm_my * acc2)

        for d in in_flight:
            d.wait_send()

    return pl.pallas_call(
        body,
        out_shape=jax.ShapeDtypeStruct((m, n), F32),
        in_specs=[pl.BlockSpec(memory_space=pltpu.VMEM)] * 5,
        out_specs=pl.BlockSpec(memory_space=pltpu.VMEM),
        scratch_shapes=[
            pltpu.VMEM((N_DEV, chunk, n), WIRE),
            pltpu.VMEM((N_DEV, chunk, n), WIRE),
            pltpu.VMEM((N_DEV, chunk, n), WIRE),
            pltpu.VMEM((N_DEV, chunk, n), WIRE),
            pltpu.VMEM((N_DEV, chunk, n), WIRE),
            pltpu.VMEM((m, n), F32),
        ] + [pltpu.SemaphoreType.DMA((N_DEV,))] * 8,
    )(ap_w, x0, mods, W_ff1, W_ff2)


def kernel(x, Wq, Wk, Wv, Wo, t_emb, W_mod, W_ff1, W_ff2):
    B, S, D = x.shape
    eps = 1e-5
    Dh = 96
    Hq = Wq.shape[1] // Dh

    mod = t_emb @ W_mod
    sa, sha, ga, sm, shm, gm = jnp.split(mod, 6, axis=-1)

    x0 = x
    mu = x0.mean(axis=-1, keepdims=True)
    var = x0.var(axis=-1, keepdims=True)
    xa = ((x0 - mu) / jnp.sqrt(var + eps)) * (1.0 + sa[:, None, :]) + sha[:, None, :]

    Q = (xa @ Wq).reshape(B, S, Hq, Dh)
    K = (xa @ Wk).reshape(B, S, Hq, Dh)
    V = (xa @ Wv).reshape(B, S, Hq, Dh)
    scores = jnp.einsum("bihd,bjhd->bhij", Q, K) * 0.10206207261596577
    p = jax.nn.softmax(scores, axis=-1)
    attn = jnp.einsum("bhij,bjhd->bihd", p, V).reshape(B, S, Hq * Dh)
    attn_partial = attn @ Wo

    mods = jnp.concatenate([ga, sm, shm, gm], axis=0)
    out = _fused_post_attn(
        attn_partial.reshape(B * S, D), x0.reshape(B * S, D), mods, W_ff1, W_ff2
    )
    return out.reshape(B, S, D)
